# baseline (speedup 1.0000x reference)
import jax
import jax.numpy as jnp
from jax import lax
from jax.experimental import pallas as pl

_N = 16384
_C = 1000
_BN = 1024
_NB = _N // _BN


def _nll_block(probt_ref, tgt_ref, w_ref, out_ref):
    t = tgt_ref[0, 0, :]
    crow = lax.broadcasted_iota(jnp.int32, (_C, _BN), 0)
    masked = jnp.where(crow == t[None, :], probt_ref[...], 0.0)
    s = jnp.sum(masked, axis=1, keepdims=True)
    out_ref[...] = jnp.sum(s * w_ref[...]).reshape(1, 1, 1)


_nll_partials = pl.pallas_call(
    _nll_block,
    grid=(_NB,),
    in_specs=[
        pl.BlockSpec((_C, _BN), lambda i: (0, i)),
        pl.BlockSpec((1, 1, _BN), lambda i: (i, 0, 0)),
        pl.BlockSpec((_C, 1), lambda i: (0, 0)),
    ],
    out_specs=pl.BlockSpec((1, 1, 1), lambda i: (i, 0, 0)),
    out_shape=jax.ShapeDtypeStruct((_NB, 1, 1), jnp.float32),
)


def kernel(prob, target, weight):
    tgt_3d = target.reshape(_NB, 1, _BN)
    partials = _nll_partials(prob.T, tgt_3d, weight.reshape(_C, 1))
    return -jnp.sum(partials)


# speedup vs baseline: 1.1284x; 1.1284x over previous
import jax
import jax.numpy as jnp
from jax import lax
from jax.experimental import pallas as pl

_N = 16384
_C = 1000
_BN = 2048
_NB = _N // _BN


def _nll_block(probt_ref, tgt_ref, w_ref, out_ref):
    t = tgt_ref[0, 0, :]
    crow = lax.broadcasted_iota(jnp.int32, (_C, _BN), 0)
    masked = jnp.where(crow == t[None, :], probt_ref[...], 0.0)
    s = jnp.sum(masked, axis=1, keepdims=True)
    out_ref[...] = jnp.sum(s * w_ref[...]).reshape(1, 1, 1)


_nll_partials = pl.pallas_call(
    _nll_block,
    grid=(_NB,),
    in_specs=[
        pl.BlockSpec((_C, _BN), lambda i: (0, i)),
        pl.BlockSpec((1, 1, _BN), lambda i: (i, 0, 0)),
        pl.BlockSpec((_C, 1), lambda i: (0, 0)),
    ],
    out_specs=pl.BlockSpec((1, 1, 1), lambda i: (i, 0, 0)),
    out_shape=jax.ShapeDtypeStruct((_NB, 1, 1), jnp.float32),
)


def kernel(prob, target, weight):
    tgt_3d = target.reshape(_NB, 1, _BN)
    partials = _nll_partials(prob.T, tgt_3d, weight.reshape(_C, 1))
    return -jnp.sum(partials)
